# initial kernel scaffold (unmeasured)
import jax
import jax.numpy as jnp
from jax import lax
from jax.experimental import pallas as pl
from jax.experimental.pallas import tpu as pltpu

N_DEV = 32


def kernel(x, w_mat, scale_x, scale_w):
    m_per, k = x.shape
    n = w_mat.shape[1]
    n_per = n // N_DEV
    m = m_per * N_DEV

    def body(x_ref, w_ref, sx_ref, sw_ref, out_ref, send_buf, send_sems, recv_sem):
        my = lax.axis_index("i")
        scale = sx_ref[0] * sw_ref[0]
        xv = x_ref[...]

        sends = []
        for t in range(N_DEV):
            j = (my + t) % N_DEV
            wj = w_ref[:, pl.ds(j * n_per, n_per)]
            acc = lax.dot_general(
                xv, wj, (((1,), (0,)), ((), ())),
                preferred_element_type=jnp.float32,
            )
            z = jnp.maximum(acc * scale, 0.0)
            if t == 0:
                out_ref[pl.ds(my * m_per, m_per), :] = z
            else:
                send_buf[t, :, :] = z
                rdma = pltpu.make_async_remote_copy(
                    src_ref=send_buf.at[t],
                    dst_ref=out_ref.at[pl.ds(my * m_per, m_per), :],
                    send_sem=send_sems.at[t],
                    recv_sem=recv_sem,
                    device_id=(j,),
                    device_id_type=pl.DeviceIdType.MESH,
                )
                rdma.start()
                sends.append(rdma)

        for rdma in sends:
            rdma.wait_send()
        recv_wait = pltpu.make_async_remote_copy(
            src_ref=send_buf.at[0],
            dst_ref=out_ref.at[pl.ds(0, m_per), :],
            send_sem=send_sems.at[0],
            recv_sem=recv_sem,
            device_id=(my,),
            device_id_type=pl.DeviceIdType.MESH,
        )
        for _ in range(N_DEV - 1):
            recv_wait.wait_recv()

    return pl.pallas_call(
        body,
        out_shape=jax.ShapeDtypeStruct((m, n_per), jnp.float32),
        in_specs=[
            pl.BlockSpec(memory_space=pltpu.VMEM),
            pl.BlockSpec(memory_space=pltpu.VMEM),
            pl.BlockSpec(memory_space=pltpu.SMEM),
            pl.BlockSpec(memory_space=pltpu.SMEM),
        ],
        out_specs=pl.BlockSpec(memory_space=pltpu.VMEM),
        scratch_shapes=[
            pltpu.VMEM((N_DEV, m_per, n_per), jnp.float32),
            pltpu.SemaphoreType.DMA((N_DEV,)),
            pltpu.SemaphoreType.DMA,
        ],
        compiler_params=pltpu.CompilerParams(
            collective_id=0,
            vmem_limit_bytes=100 * 1024 * 1024,
        ),
    )(x, w_mat, scale_x, scale_w)


# baseline (device time: 92649 ns/iter reference)
import jax
import jax.numpy as jnp
from jax import lax
from jax.experimental import pallas as pl
from jax.experimental.pallas import tpu as pltpu

N_DEV = 32


def kernel(x, w_mat, scale_x, scale_w):
    m_per, k = x.shape
    n = w_mat.shape[1]
    n_per = n // N_DEV
    m = m_per * N_DEV

    def body(x_ref, w_hbm, sx_ref, sw_ref, out_ref,
             wbuf, send_buf, copy_sems, send_sems, recv_sem):
        my = lax.axis_index("i")
        scale = sx_ref[0] * sw_ref[0]
        x8 = x_ref[...].astype(jnp.float8_e5m2)

        def start_copy(t):
            j = (my + t) % N_DEV
            cp = pltpu.make_async_copy(
                w_hbm.at[:, pl.ds(j * n_per, n_per)],
                wbuf.at[t % 2],
                copy_sems.at[t % 2],
            )
            cp.start()
            return cp

        cp = start_copy(0)
        sends = []
        for t in range(N_DEV):
            j = (my + t) % N_DEV
            cp.wait()
            if t + 1 < N_DEV:
                cp = start_copy(t + 1)
            w8 = wbuf[t % 2].astype(jnp.float8_e5m2)
            acc = lax.dot_general(
                x8, w8, (((1,), (0,)), ((), ())),
                preferred_element_type=jnp.float32,
            )
            z = jnp.maximum(acc * scale, 0.0)
            if t == 0:
                out_ref[pl.ds(my * m_per, m_per), :] = z
            else:
                send_buf[t, :, :] = z
                rdma = pltpu.make_async_remote_copy(
                    src_ref=send_buf.at[t],
                    dst_ref=out_ref.at[pl.ds(my * m_per, m_per), :],
                    send_sem=send_sems.at[t],
                    recv_sem=recv_sem,
                    device_id=(j,),
                    device_id_type=pl.DeviceIdType.MESH,
                )
                rdma.start()
                sends.append(rdma)

        for rdma in sends:
            rdma.wait_send()
        recv_wait = pltpu.make_async_remote_copy(
            src_ref=send_buf.at[0],
            dst_ref=out_ref.at[pl.ds(0, m_per), :],
            send_sem=send_sems.at[0],
            recv_sem=recv_sem,
            device_id=(my,),
            device_id_type=pl.DeviceIdType.MESH,
        )
        for _ in range(N_DEV - 1):
            recv_wait.wait_recv()

    return pl.pallas_call(
        body,
        out_shape=jax.ShapeDtypeStruct((m, n_per), jnp.float32),
        in_specs=[
            pl.BlockSpec(memory_space=pltpu.VMEM),
            pl.BlockSpec(memory_space=pltpu.MemorySpace.HBM),
            pl.BlockSpec(memory_space=pltpu.SMEM),
            pl.BlockSpec(memory_space=pltpu.SMEM),
        ],
        out_specs=pl.BlockSpec(memory_space=pltpu.VMEM),
        scratch_shapes=[
            pltpu.VMEM((2, k, n_per), jnp.float32),
            pltpu.VMEM((N_DEV, m_per, n_per), jnp.float32),
            pltpu.SemaphoreType.DMA((2,)),
            pltpu.SemaphoreType.DMA((N_DEV,)),
            pltpu.SemaphoreType.DMA,
        ],
        compiler_params=pltpu.CompilerParams(
            vmem_limit_bytes=100 * 1024 * 1024,
        ),
    )(x, w_mat, scale_x, scale_w)


# device time: 78268 ns/iter; 1.1837x vs baseline; 1.1837x over previous
import jax
import jax.numpy as jnp
from jax import lax
from jax.experimental import pallas as pl
from jax.experimental.pallas import tpu as pltpu

N_DEV = 32
NBUF = 4


def kernel(x, w_mat, scale_x, scale_w):
    m_per, k = x.shape
    n = w_mat.shape[1]
    n_per = n // N_DEV
    m = m_per * N_DEV

    def body(x_ref, w_hbm, sx_ref, sw_ref, out_ref,
             wbuf, send_buf, copy_sems, send_sems, recv_sem):
        my = lax.axis_index("i")
        scale = sx_ref[0] * sw_ref[0]
        x8 = x_ref[...].astype(jnp.float8_e5m2)

        def start_copy(t):
            j = (my + t) % N_DEV
            cp = pltpu.make_async_copy(
                w_hbm.at[:, pl.ds(j * n_per, n_per)],
                wbuf.at[t % NBUF],
                copy_sems.at[t % NBUF],
            )
            cp.start()
            return cp

        cps = [start_copy(t) for t in range(NBUF - 1)]
        sends = []
        for t in range(N_DEV):
            j = (my + t) % N_DEV
            if t + NBUF - 1 < N_DEV:
                cps.append(start_copy(t + NBUF - 1))
            cps.pop(0).wait()
            w8 = wbuf[t % NBUF].astype(jnp.float8_e5m2)
            acc = lax.dot_general(
                x8, w8, (((1,), (0,)), ((), ())),
                preferred_element_type=jnp.float32,
            )
            z = jnp.maximum(acc * scale, 0.0)
            if t == 0:
                out_ref[pl.ds(my * m_per, m_per), :] = z
            else:
                send_buf[t, :, :] = z
                rdma = pltpu.make_async_remote_copy(
                    src_ref=send_buf.at[t],
                    dst_ref=out_ref.at[pl.ds(my * m_per, m_per), :],
                    send_sem=send_sems.at[t],
                    recv_sem=recv_sem,
                    device_id=(j,),
                    device_id_type=pl.DeviceIdType.MESH,
                )
                rdma.start()
                sends.append(rdma)

        for rdma in sends:
            rdma.wait_send()
        recv_wait = pltpu.make_async_remote_copy(
            src_ref=send_buf.at[0],
            dst_ref=out_ref.at[pl.ds(0, m_per), :],
            send_sem=send_sems.at[0],
            recv_sem=recv_sem,
            device_id=(my,),
            device_id_type=pl.DeviceIdType.MESH,
        )
        for _ in range(N_DEV - 1):
            recv_wait.wait_recv()

    return pl.pallas_call(
        body,
        out_shape=jax.ShapeDtypeStruct((m, n_per), jnp.float32),
        in_specs=[
            pl.BlockSpec(memory_space=pltpu.VMEM),
            pl.BlockSpec(memory_space=pltpu.MemorySpace.HBM),
            pl.BlockSpec(memory_space=pltpu.SMEM),
            pl.BlockSpec(memory_space=pltpu.SMEM),
        ],
        out_specs=pl.BlockSpec(memory_space=pltpu.VMEM),
        scratch_shapes=[
            pltpu.VMEM((NBUF, k, n_per), jnp.float32),
            pltpu.VMEM((N_DEV, m_per, n_per), jnp.float32),
            pltpu.SemaphoreType.DMA((NBUF,)),
            pltpu.SemaphoreType.DMA((N_DEV,)),
            pltpu.SemaphoreType.DMA,
        ],
        compiler_params=pltpu.CompilerParams(
            vmem_limit_bytes=100 * 1024 * 1024,
        ),
    )(x, w_mat, scale_x, scale_w)
